# baseline (speedup 1.0000x reference)
import functools

import jax
import jax.numpy as jnp
from jax import lax
from jax.experimental import pallas as pl
from jax.experimental.pallas import tpu as pltpu
from jax.experimental.pallas import tpu_sc as plsc

B, F, H, W, C = 8, 8, 48, 48, 256
WC = W * C
EPS = 1e-12
NCORES, NSUB = 2, 16
NW = NCORES * NSUB
NB_TC = 8
NB_SC = B - NB_TC
VPT = C // 16
ROWS_PER_BLOCK = 16


def _tc_body(g_ref, rp_ref, cp_ref, tt_ref, gm_ref, bt_ref, o_ref, acc_ref):
    f = pl.program_id(2)
    g = g_ref[0, 0]

    @pl.when(f == 0)
    def _():
        acc_ref[...] = g

    @pl.when(f > 0)
    def _():
        acc_ref[...] = acc_ref[...] + g

    @pl.when(f == F - 1)
    def _():
        s = acc_ref[...] * (1.0 / F)
        pos = rp_ref[...][:, None, :] + cp_ref[...][None, :, :]
        e = s + pos + tt_ref[...][0][None, None, :]
        mean = jnp.mean(e, axis=-1, keepdims=True)
        d = e - mean
        var = jnp.mean(d * d, axis=-1, keepdims=True)
        o_ref[0] = d * lax.rsqrt(var + EPS) * gm_ref[0] + bt_ref[0]


def _tc_call(grid, row_pos, col_pos, token_type, ln_gamma, ln_beta):
    nrow = H // ROWS_PER_BLOCK
    return pl.pallas_call(
        _tc_body,
        grid=(NB_TC, nrow, F),
        in_specs=[
            pl.BlockSpec((1, 1, ROWS_PER_BLOCK, W, C),
                         lambda b, i, f: (b, f, i, 0, 0)),
            pl.BlockSpec((ROWS_PER_BLOCK, C), lambda b, i, f: (i, 0)),
            pl.BlockSpec((W, C), lambda b, i, f: (0, 0)),
            pl.BlockSpec((1, C), lambda b, i, f: (0, 0)),
            pl.BlockSpec((1, C), lambda b, i, f: (0, 0)),
            pl.BlockSpec((1, C), lambda b, i, f: (0, 0)),
        ],
        out_specs=pl.BlockSpec((1, ROWS_PER_BLOCK, W, C),
                               lambda b, i, f: (b, i, 0, 0)),
        out_shape=jax.ShapeDtypeStruct((NB_TC, H, W, C), jnp.float32),
        scratch_shapes=[pltpu.VMEM((ROWS_PER_BLOCK, W, C), jnp.float32)],
    )(grid, row_pos, col_pos, token_type,
      ln_gamma.reshape(1, C), ln_beta.reshape(1, C))


_GDN = lax.GatherDimensionNumbers(
    offset_dims=(), collapsed_slice_dims=(0,), start_index_map=(0,))


def _perm16(x, idx):
    return lax.gather(x, idx[:, None], _GDN, (1,),
                      mode=lax.GatherScatterMode.PROMISE_IN_BOUNDS)


def _lanesum(x):
    ii = lax.iota(jnp.int32, 16)
    for sh in (8, 4, 2, 1):
        x = x + _perm16(x, ii ^ sh)
    return x


def _rsqrt16(v):
    i = lax.bitcast_convert_type(v, jnp.int32)
    y = lax.bitcast_convert_type(jnp.int32(0x5F3759DF) - (i >> 1), jnp.float32)
    for _ in range(3):
        y = y * (1.5 - 0.5 * v * y * y)
    return y


NHF = 2
TPC = W // NHF
CWC = TPC * C


def _sc_body(grid_h, row1, col1, tt_h, gm_h, bt_h, out_h,
             fbuf, obuf, colv, ttv, gmv, btv, rowv, rowtt,
             isem0, isem1, osem0, osem1):
    cid = lax.axis_index("c")
    sid = lax.axis_index("s")
    wid = cid * NSUB + sid
    pltpu.sync_copy(col1, colv)
    pltpu.sync_copy(tt_h, ttv)
    pltpu.sync_copy(gm_h, gmv)
    pltpu.sync_copy(bt_h, btv)
    isems = (isem0, isem1)
    osems = (osem0, osem1)

    nchunk = NB_SC * H * NHF
    npw = nchunk // NW

    def cidx(i):
        g = wid + i * NW
        b = g // (H * NHF)
        r = g - b * (H * NHF)
        h = r // NHF
        hf = r - h * NHF
        return b, h, hf

    def fire(i, p):
        b, h, hf = cidx(i)
        bb = NB_TC + b
        w0 = hf * TPC
        for f in range(F):
            pltpu.async_copy(grid_h.at[bb, f, h, pl.ds(w0, TPC)],
                             fbuf.at[p, f], isems[p])
        pltpu.async_copy(row1.at[pl.ds(h * C, C)], rowv.at[p], isems[p])

    def wait_in(p):
        for f in range(F):
            pltpu.make_async_copy(grid_h.at[0, 0, 0, pl.ds(0, TPC)],
                                  fbuf.at[p, f], isems[p]).wait()
        pltpu.make_async_copy(row1.at[pl.ds(0, C)], rowv.at[p], isems[p]).wait()

    def compute(i, p):
        b, h, hf = cidx(i)
        hfo = hf * CWC
        for v in range(VPT):
            o = v * 16
            rowtt[p, pl.ds(o, 16)] = rowv[p, pl.ds(o, 16)] + ttv[pl.ds(o, 16)]

        def tok(j, carry2):
            bc = hfo + j * C
            es = []
            for v in range(VPT):
                x = fbuf[p, 0, j, pl.ds(v * 16, 16)]
                for f in range(1, F):
                    x = x + fbuf[p, f, j, pl.ds(v * 16, 16)]
                x = (x * (1.0 / F) + rowtt[p, pl.ds(v * 16, 16)]
                     + colv[pl.ds(bc + v * 16, 16)])
                es.append(x)
            tot = es[0]
            for v in range(1, VPT):
                tot = tot + es[v]
            mv = _lanesum(tot) * (1.0 / C)
            ds = [e - mv for e in es]
            sq = ds[0] * ds[0]
            for v in range(1, VPT):
                sq = sq + ds[v] * ds[v]
            var = _lanesum(sq) * (1.0 / C)
            rstd = _rsqrt16(var + EPS)
            for v in range(VPT):
                obuf[p, j, pl.ds(v * 16, 16)] = (
                    ds[v] * rstd * gmv[pl.ds(v * 16, 16)]
                    + btv[pl.ds(v * 16, 16)])
            return carry2

        lax.fori_loop(0, TPC, tok, 0)
        pltpu.async_copy(obuf.at[p],
                         out_h.at[b, h, pl.ds(hf * TPC, TPC)], osems[p])

    fire(0, 0)

    def pair(pi, carry):
        for p in range(2):
            ci = pi * 2 + p

            @pl.when(ci + 1 < npw)
            def _():
                fire(ci + 1, 1 - p)

            wait_in(p)

            @pl.when(ci >= 2)
            def _():
                pltpu.make_async_copy(obuf.at[p],
                                      out_h.at[0, 0, pl.ds(0, TPC)],
                                      osems[p]).wait()

            compute(ci, p)
        return carry

    lax.fori_loop(0, npw // 2, pair, 0)
    if npw % 2:
        ci = npw - 1
        wait_in(0)
        if npw > 2:
            pltpu.make_async_copy(obuf.at[0],
                                  out_h.at[0, 0, pl.ds(0, TPC)],
                                  osems[0]).wait()
        compute(ci, 0)
    for p in range(2):
        pltpu.make_async_copy(obuf.at[p],
                              out_h.at[0, 0, pl.ds(0, TPC)],
                              osems[p]).wait()


def _sc_call(grid, row_pos, col_pos, token_type, ln_gamma, ln_beta):
    mesh = plsc.VectorSubcoreMesh(core_axis_name="c", subcore_axis_name="s")
    k = functools.partial(
        pl.kernel, mesh=mesh,
        out_type=jax.ShapeDtypeStruct((NB_SC, H, W, C), jnp.float32),
        compiler_params=pltpu.CompilerParams(use_tc_tiling_on_sc=True),
        scratch_types=[
            pltpu.VMEM((2, F, TPC, C), jnp.float32),
            pltpu.VMEM((2, TPC, C), jnp.float32),
            pltpu.VMEM((WC,), jnp.float32),
            pltpu.VMEM((C,), jnp.float32),
            pltpu.VMEM((C,), jnp.float32),
            pltpu.VMEM((C,), jnp.float32),
            pltpu.VMEM((2, C), jnp.float32),
            pltpu.VMEM((2, C), jnp.float32),
            pltpu.SemaphoreType.DMA,
            pltpu.SemaphoreType.DMA,
            pltpu.SemaphoreType.DMA,
            pltpu.SemaphoreType.DMA,
        ],
    )(_sc_body)
    return k(grid,
             row_pos.reshape(row_pos.shape[0] * C),
             col_pos[:W].reshape(WC),
             token_type.reshape(C),
             ln_gamma,
             ln_beta)


def kernel(grid, row_pos, col_pos, token_type, ln_gamma, ln_beta):
    parts = []
    if NB_TC:
        tc_out = _tc_call(grid, row_pos, col_pos, token_type, ln_gamma, ln_beta)
        parts.append(tc_out.reshape(NB_TC, H * W, C))
    if NB_SC:
        sc_out = _sc_call(grid, row_pos, col_pos, token_type, ln_gamma, ln_beta)
        parts.append(sc_out.reshape(NB_SC, H * W, C))
    emb = parts[0] if len(parts) == 1 else jnp.concatenate(parts, axis=0)
    sampled_indices = jnp.arange(H * W, dtype=jnp.int32)
    return (emb, sampled_indices)


# speedup vs baseline: 2.9315x; 2.9315x over previous
import functools

import jax
import jax.numpy as jnp
from jax import lax
from jax.experimental import pallas as pl
from jax.experimental.pallas import tpu as pltpu
from jax.experimental.pallas import tpu_sc as plsc

B, F, H, W, C = 8, 8, 48, 48, 256
WC = W * C
EPS = 1e-12
NCORES, NSUB = 2, 16
NW = NCORES * NSUB
NB_TC = 8
NB_SC = B - NB_TC
VPT = C // 16
ROWS_PER_BLOCK = 16


def _tc_body(g0_ref, g1_ref, rp_ref, cp_ref, tt_ref, gm_ref, bt_ref, o_ref):
    s = ((jnp.sum(g0_ref[0], axis=0) + jnp.sum(g1_ref[0], axis=0))
         * (1.0 / F))
    pos = rp_ref[...][:, None, :] + cp_ref[...][None, :, :]
    e = s + pos + tt_ref[...][0][None, None, :]
    mean = jnp.mean(e, axis=-1, keepdims=True)
    d = e - mean
    var = jnp.mean(d * d, axis=-1, keepdims=True)
    o_ref[0] = d * lax.rsqrt(var + EPS) * gm_ref[0] + bt_ref[0]


def _tc_call(grid, row_pos, col_pos, token_type, ln_gamma, ln_beta):
    nrow = H // ROWS_PER_BLOCK
    hf = F // 2
    return pl.pallas_call(
        _tc_body,
        grid=(NB_TC, nrow),
        in_specs=[
            pl.BlockSpec((1, hf, ROWS_PER_BLOCK, W, C),
                         lambda b, i: (b, 0, i, 0, 0)),
            pl.BlockSpec((1, hf, ROWS_PER_BLOCK, W, C),
                         lambda b, i: (b, 1, i, 0, 0)),
            pl.BlockSpec((ROWS_PER_BLOCK, C), lambda b, i: (i, 0)),
            pl.BlockSpec((W, C), lambda b, i: (0, 0)),
            pl.BlockSpec((1, C), lambda b, i: (0, 0)),
            pl.BlockSpec((1, C), lambda b, i: (0, 0)),
            pl.BlockSpec((1, C), lambda b, i: (0, 0)),
        ],
        out_specs=pl.BlockSpec((1, ROWS_PER_BLOCK, W, C), lambda b, i: (b, i, 0, 0)),
        out_shape=jax.ShapeDtypeStruct((NB_TC, H, W, C), jnp.float32),
    )(grid, grid, row_pos, col_pos, token_type,
      ln_gamma.reshape(1, C), ln_beta.reshape(1, C))


_GDN = lax.GatherDimensionNumbers(
    offset_dims=(), collapsed_slice_dims=(0,), start_index_map=(0,))


def _perm16(x, idx):
    return lax.gather(x, idx[:, None], _GDN, (1,),
                      mode=lax.GatherScatterMode.PROMISE_IN_BOUNDS)


def _lanesum(x):
    ii = lax.iota(jnp.int32, 16)
    for sh in (8, 4, 2, 1):
        x = x + _perm16(x, ii ^ sh)
    return x


def _rsqrt16(v):
    i = lax.bitcast_convert_type(v, jnp.int32)
    y = lax.bitcast_convert_type(jnp.int32(0x5F3759DF) - (i >> 1), jnp.float32)
    for _ in range(3):
        y = y * (1.5 - 0.5 * v * y * y)
    return y


NHF = 2
TPC = W // NHF
CWC = TPC * C


def _sc_body(grid_h, row1, col1, tt_h, gm_h, bt_h, out_h,
             fbuf, obuf, colv, ttv, gmv, btv, rowv, rowtt,
             isem0, isem1, osem0, osem1):
    cid = lax.axis_index("c")
    sid = lax.axis_index("s")
    wid = cid * NSUB + sid
    pltpu.sync_copy(col1, colv)
    pltpu.sync_copy(tt_h, ttv)
    pltpu.sync_copy(gm_h, gmv)
    pltpu.sync_copy(bt_h, btv)
    isems = (isem0, isem1)
    osems = (osem0, osem1)

    nchunk = NB_SC * H * NHF
    npw = nchunk // NW

    def cidx(i):
        g = wid + i * NW
        b = g // (H * NHF)
        r = g - b * (H * NHF)
        h = r // NHF
        hf = r - h * NHF
        return b, h, hf

    def fire(i, p):
        b, h, hf = cidx(i)
        bb = NB_TC + b
        w0 = hf * TPC
        for f in range(F):
            pltpu.async_copy(grid_h.at[bb, f, h, pl.ds(w0, TPC)],
                             fbuf.at[p, f], isems[p])
        pltpu.async_copy(row1.at[pl.ds(h * C, C)], rowv.at[p], isems[p])

    def wait_in(p):
        for f in range(F):
            pltpu.make_async_copy(grid_h.at[0, 0, 0, pl.ds(0, TPC)],
                                  fbuf.at[p, f], isems[p]).wait()
        pltpu.make_async_copy(row1.at[pl.ds(0, C)], rowv.at[p], isems[p]).wait()

    def compute(i, p):
        b, h, hf = cidx(i)
        hfo = hf * CWC
        for v in range(VPT):
            o = v * 16
            rowtt[p, pl.ds(o, 16)] = rowv[p, pl.ds(o, 16)] + ttv[pl.ds(o, 16)]

        def tok(j, carry2):
            bc = hfo + j * C
            es = []
            for v in range(VPT):
                x = fbuf[p, 0, j, pl.ds(v * 16, 16)]
                for f in range(1, F):
                    x = x + fbuf[p, f, j, pl.ds(v * 16, 16)]
                x = (x * (1.0 / F) + rowtt[p, pl.ds(v * 16, 16)]
                     + colv[pl.ds(bc + v * 16, 16)])
                es.append(x)
            tot = es[0]
            for v in range(1, VPT):
                tot = tot + es[v]
            mv = _lanesum(tot) * (1.0 / C)
            ds = [e - mv for e in es]
            sq = ds[0] * ds[0]
            for v in range(1, VPT):
                sq = sq + ds[v] * ds[v]
            var = _lanesum(sq) * (1.0 / C)
            rstd = _rsqrt16(var + EPS)
            for v in range(VPT):
                obuf[p, j, pl.ds(v * 16, 16)] = (
                    ds[v] * rstd * gmv[pl.ds(v * 16, 16)]
                    + btv[pl.ds(v * 16, 16)])
            return carry2

        lax.fori_loop(0, TPC, tok, 0)
        pltpu.async_copy(obuf.at[p],
                         out_h.at[b, h, pl.ds(hf * TPC, TPC)], osems[p])

    fire(0, 0)

    def pair(pi, carry):
        for p in range(2):
            ci = pi * 2 + p

            @pl.when(ci + 1 < npw)
            def _():
                fire(ci + 1, 1 - p)

            wait_in(p)

            @pl.when(ci >= 2)
            def _():
                pltpu.make_async_copy(obuf.at[p],
                                      out_h.at[0, 0, pl.ds(0, TPC)],
                                      osems[p]).wait()

            compute(ci, p)
        return carry

    lax.fori_loop(0, npw // 2, pair, 0)
    if npw % 2:
        ci = npw - 1
        wait_in(0)
        if npw > 2:
            pltpu.make_async_copy(obuf.at[0],
                                  out_h.at[0, 0, pl.ds(0, TPC)],
                                  osems[0]).wait()
        compute(ci, 0)
    for p in range(2):
        pltpu.make_async_copy(obuf.at[p],
                              out_h.at[0, 0, pl.ds(0, TPC)],
                              osems[p]).wait()


def _sc_call(grid, row_pos, col_pos, token_type, ln_gamma, ln_beta):
    mesh = plsc.VectorSubcoreMesh(core_axis_name="c", subcore_axis_name="s")
    k = functools.partial(
        pl.kernel, mesh=mesh,
        out_type=jax.ShapeDtypeStruct((NB_SC, H, W, C), jnp.float32),
        compiler_params=pltpu.CompilerParams(use_tc_tiling_on_sc=True),
        scratch_types=[
            pltpu.VMEM((2, F, TPC, C), jnp.float32),
            pltpu.VMEM((2, TPC, C), jnp.float32),
            pltpu.VMEM((WC,), jnp.float32),
            pltpu.VMEM((C,), jnp.float32),
            pltpu.VMEM((C,), jnp.float32),
            pltpu.VMEM((C,), jnp.float32),
            pltpu.VMEM((2, C), jnp.float32),
            pltpu.VMEM((2, C), jnp.float32),
            pltpu.SemaphoreType.DMA,
            pltpu.SemaphoreType.DMA,
            pltpu.SemaphoreType.DMA,
            pltpu.SemaphoreType.DMA,
        ],
    )(_sc_body)
    return k(grid,
             row_pos.reshape(row_pos.shape[0] * C),
             col_pos[:W].reshape(WC),
             token_type.reshape(C),
             ln_gamma,
             ln_beta)


def kernel(grid, row_pos, col_pos, token_type, ln_gamma, ln_beta):
    parts = []
    if NB_TC:
        tc_out = _tc_call(grid, row_pos, col_pos, token_type, ln_gamma, ln_beta)
        parts.append(tc_out.reshape(NB_TC, H * W, C))
    if NB_SC:
        sc_out = _sc_call(grid, row_pos, col_pos, token_type, ln_gamma, ln_beta)
        parts.append(sc_out.reshape(NB_SC, H * W, C))
    emb = parts[0] if len(parts) == 1 else jnp.concatenate(parts, axis=0)
    sampled_indices = jnp.arange(H * W, dtype=jnp.int32)
    return (emb, sampled_indices)
